# baseline (speedup 1.0000x reference)
import functools

import jax
import jax.numpy as jnp
from jax.experimental import pallas as pl
from jax.experimental.pallas import tpu as pltpu

B, P, Q, E = 16, 4096, 64, 32
BP = 1024
NP = P // BP

_CONTRACT0 = (((0,), (0,)), ((), ()))


def _softplus(x):
    return jnp.maximum(x, 0.0) + jnp.log1p(jnp.exp(-jnp.abs(x)))


def _cost_kernel(pl_ref, px_ref, py_ref, tx_ref, ty_ref,
                 ml_ref, mp_ref, sv_ref, sp_ref, out_ref,
                 acc_pt, acc_nt, acc_num, acc_neg, acc_denq, acc_dene, acc_nnz):
    k = pl.program_id(1)

    @pl.when(k == 0)
    def _init():
        acc_pt[...] = jnp.zeros_like(acc_pt)
        acc_nt[...] = jnp.zeros_like(acc_nt)
        acc_num[...] = jnp.zeros_like(acc_num)
        acc_neg[...] = jnp.zeros_like(acc_neg)
        acc_denq[...] = jnp.zeros_like(acc_denq)
        acc_dene[...] = jnp.zeros_like(acc_dene)
        acc_nnz[...] = jnp.zeros_like(acc_nnz)

    x = ml_ref[0]
    m = mp_ref[0]
    sv = sv_ref[0]
    targ = sp_ref[0]

    l = jnp.log1p(jnp.exp(-jnp.abs(x)))
    pos = (jnp.maximum(-x, 0.0) + l) * m
    neg = (jnp.maximum(x, 0.0) + l) * m

    masked = jnp.where(m > 0.0, x, -1e30)
    mx = jnp.max(masked, axis=1, keepdims=True)
    ex = jnp.exp(masked - mx) * m
    s = jnp.sum(ex, axis=1, keepdims=True)
    portions = ex / jnp.maximum(s, 1e-12)

    dot = functools.partial(jax.lax.dot_general, dimension_numbers=_CONTRACT0,
                            preferred_element_type=jnp.float32)
    ones = jnp.ones((x.shape[0], 1), jnp.float32)
    acc_pt[...] += dot(pos, targ)
    acc_nt[...] += dot(neg, targ)
    acc_num[...] += dot(portions, sv)
    acc_neg[...] += dot(neg, ones)
    acc_denq[...] += dot(portions, ones)
    acc_dene[...] += jnp.sum(sv, axis=0, keepdims=True)
    acc_nnz[...] += jnp.sum(targ, axis=(0, 1), keepdims=True)

    @pl.when(k == NP - 1)
    def _finalize():
        nnz = jnp.maximum(acc_nnz[...], 1.0)
        mask_cost = (acc_pt[...] + acc_neg[...] - acc_nt[...]) / nnz
        den = acc_denq[...] + acc_dene[...] + 1.0
        dice_cost = 1.0 - (2.0 * acc_num[...] + 1.0) / den
        cls = _softplus(-pl_ref[0])
        dx = px_ref[0] - tx_ref[0]
        dy = py_ref[0] - ty_ref[0]
        adx = jnp.abs(dx)
        ady = jnp.abs(dy)
        hx = jnp.where(adx < 1.0, 0.5 * dx * dx, adx - 0.5)
        hy = jnp.where(ady < 1.0, 0.5 * dy * dy, ady - 0.5)
        out_ref[0] = cls + mask_cost + dice_cost + 0.5 * (hx + hy)


@jax.jit
def kernel(pred_logits, mask_logits, mask_present, segmap_values, segmap_present,
           pred_positions, true_positions, query_batch_offsets, electron_batch_offsets):
    del query_batch_offsets, electron_batch_offsets
    pl3 = pred_logits.reshape(B, Q, 1)
    pp = pred_positions.reshape(B, Q, 2)
    tp = true_positions.reshape(B, E, 2)
    px = pp[:, :, 0:1]
    py = pp[:, :, 1:2]
    tx = tp[:, :, 0].reshape(B, 1, E)
    ty = tp[:, :, 1].reshape(B, 1, E)

    grid = (B, NP)
    qe = pl.BlockSpec((1, Q, E), lambda b, k: (b, 0, 0))
    per_b_q1 = pl.BlockSpec((1, Q, 1), lambda b, k: (b, 0, 0))
    per_b_1e = pl.BlockSpec((1, 1, E), lambda b, k: (b, 0, 0))
    pq = pl.BlockSpec((1, BP, Q), lambda b, k: (b, k, 0))
    pe = pl.BlockSpec((1, BP, E), lambda b, k: (b, k, 0))

    return pl.pallas_call(
        _cost_kernel,
        grid=grid,
        in_specs=[per_b_q1, per_b_q1, per_b_q1, per_b_1e, per_b_1e, pq, pq, pe, pe],
        out_specs=qe,
        out_shape=jax.ShapeDtypeStruct((B, Q, E), jnp.float32),
        scratch_shapes=[
            pltpu.VMEM((Q, E), jnp.float32),
            pltpu.VMEM((Q, E), jnp.float32),
            pltpu.VMEM((Q, E), jnp.float32),
            pltpu.VMEM((Q, 1), jnp.float32),
            pltpu.VMEM((Q, 1), jnp.float32),
            pltpu.VMEM((1, E), jnp.float32),
            pltpu.VMEM((1, 1), jnp.float32),
        ],
    )(pl3, px, py, tx, ty, mask_logits, mask_present, segmap_values, segmap_present)


# speedup vs baseline: 1.0872x; 1.0872x over previous
import functools

import jax
import jax.numpy as jnp
from jax.experimental import pallas as pl
from jax.experimental.pallas import tpu as pltpu

B, P, Q, E = 16, 4096, 64, 32

_CONTRACT0 = (((0,), (0,)), ((), ()))


def _cost_kernel(pl_ref, px_ref, py_ref, tx_ref, ty_ref,
                 ml_ref, mp_ref, sv_ref, sp_ref, out_ref):
    x = ml_ref[0]
    m = mp_ref[0]
    sv = sv_ref[0]
    targ = sp_ref[0]

    l = jnp.log1p(jnp.exp(-jnp.abs(x)))
    xm = x * m
    negm = (jnp.maximum(-x, 0.0) + l) * m + xm

    masked = jnp.where(m > 0.0, x, -1e30)
    mx = jnp.max(masked, axis=1, keepdims=True)
    ex = jnp.exp(masked - mx) * m
    s = jnp.sum(ex, axis=1, keepdims=True)
    portions = ex / jnp.maximum(s, 1e-12)

    dot = functools.partial(jax.lax.dot_general, dimension_numbers=_CONTRACT0,
                            preferred_element_type=jnp.float32)
    ones = jnp.ones((P, 1), jnp.float32)
    xmt = dot(xm, targ)
    num = dot(portions, sv)
    negsum = dot(negm, ones)
    denq = dot(portions, ones)
    dene = dot(ones, sv)
    nnz_row = dot(ones, targ)

    nnz = jnp.maximum(jnp.sum(nnz_row), 1.0)
    mask_cost = (negsum - xmt) / nnz
    den = denq + dene + 1.0
    dice_cost = 1.0 - (2.0 * num + 1.0) / den
    pl0 = pl_ref[0]
    cls = jnp.maximum(pl0, 0.0) + jnp.log1p(jnp.exp(-jnp.abs(pl0)))
    dx = px_ref[0] - tx_ref[0]
    dy = py_ref[0] - ty_ref[0]
    adx = jnp.abs(dx)
    ady = jnp.abs(dy)
    hx = jnp.where(adx < 1.0, 0.5 * dx * dx, adx - 0.5)
    hy = jnp.where(ady < 1.0, 0.5 * dy * dy, ady - 0.5)
    out_ref[0] = cls + mask_cost + dice_cost + 0.5 * (hx + hy)


@jax.jit
def kernel(pred_logits, mask_logits, mask_present, segmap_values, segmap_present,
           pred_positions, true_positions, query_batch_offsets, electron_batch_offsets):
    del query_batch_offsets, electron_batch_offsets
    pl3 = (-pred_logits).reshape(B, Q, 1)
    pp = pred_positions.reshape(B, Q, 2)
    tp = true_positions.reshape(B, E, 2)
    px = pp[:, :, 0:1]
    py = pp[:, :, 1:2]
    tx = tp[:, :, 0].reshape(B, 1, E)
    ty = tp[:, :, 1].reshape(B, 1, E)

    qe = pl.BlockSpec((1, Q, E), lambda b: (b, 0, 0))
    per_b_q1 = pl.BlockSpec((1, Q, 1), lambda b: (b, 0, 0))
    per_b_1e = pl.BlockSpec((1, 1, E), lambda b: (b, 0, 0))
    pq = pl.BlockSpec((1, P, Q), lambda b: (b, 0, 0))
    pe = pl.BlockSpec((1, P, E), lambda b: (b, 0, 0))

    return pl.pallas_call(
        _cost_kernel,
        grid=(B,),
        in_specs=[per_b_q1, per_b_q1, per_b_q1, per_b_1e, per_b_1e, pq, pq, pe, pe],
        out_specs=qe,
        out_shape=jax.ShapeDtypeStruct((B, Q, E), jnp.float32),
        compiler_params=pltpu.CompilerParams(
            dimension_semantics=("parallel",),
        ),
    )(pl3, px, py, tx, ty, mask_logits, mask_present, segmap_values, segmap_present)
